# baseline (speedup 1.0000x reference)
import functools

import jax
import jax.numpy as jnp
from jax import lax
from jax.experimental import pallas as pl
from jax.experimental.pallas import tpu as pltpu
from jax.experimental.pallas import tpu_sc as plsc

N = 10000
E = 160000
G = 8
D = 300
NB = 25
BLK = N // NB

DPB = 512
DPI = 256
DAI = 160
NW = 32
WPD = 320
NPAD = NW * WPD
DUST = WPD
CH = 2000
NCH = E // CH
SUB = 64
CAP = CH + SUB


def _rows(i):
    return (i, 0)


def _full(i):
    return (0, 0)


def _project_body(x_ref, w0, b0, w1, b1, w2, b2, o_ref):
    h = jnp.maximum(x_ref[...] @ w0[...] + b0[...], 0.0)
    h = jnp.maximum(h @ w1[...] + b1[...], 0.0)
    o_ref[...] = h @ w2[...] + b2[...]


def _project(x, p):
    (w0, b0), (w1, b1), (w2, b2) = p
    return pl.pallas_call(
        _project_body,
        grid=(NB,),
        in_specs=[
            pl.BlockSpec((BLK, 3), _rows),
            pl.BlockSpec((3, 64), _full), pl.BlockSpec((64,), lambda i: (0,)),
            pl.BlockSpec((64, 128), _full), pl.BlockSpec((128,), lambda i: (0,)),
            pl.BlockSpec((128, D), _full), pl.BlockSpec((D,), lambda i: (0,)),
        ],
        out_specs=pl.BlockSpec((BLK, D), _rows),
        out_shape=jax.ShapeDtypeStruct((N, D), jnp.float32),
    )(x, w0, b0, w1, b1, w2, b2)


def _pre_body(h_ref, pos_ref, wh, bh, wfp, wfx, s_ref, t_ref):
    h = h_ref[...]
    pos = pos_ref[...]
    delta = h @ wh[...] + bh[...]
    s = pos @ wfp[...] + h @ wfx[...]
    s_ref[...] = jnp.concatenate(
        [s, jnp.zeros((s.shape[0], DPB - D), jnp.float32)],
        axis=1).astype(jnp.bfloat16)
    t_ref[...] = (delta - pos) @ wfp[...]


def _conv_pre(h, pos, p):
    (wh, bh), = p['h']
    (wf, _bf), = p['f']
    wfp = wf[:3]
    wfx = wf[3:]
    return pl.pallas_call(
        _pre_body,
        grid=(NB,),
        in_specs=[
            pl.BlockSpec((BLK, D), _rows),
            pl.BlockSpec((BLK, 3), _rows),
            pl.BlockSpec((D, 3), _full), pl.BlockSpec((3,), lambda i: (0,)),
            pl.BlockSpec((3, D), _full),
            pl.BlockSpec((D, D), _full),
        ],
        out_specs=[pl.BlockSpec((BLK, DPB), _rows), pl.BlockSpec((BLK, D), _rows)],
        out_shape=[jax.ShapeDtypeStruct((N, DPB), jnp.bfloat16),
                   jax.ShapeDtypeStruct((N, D), jnp.float32)],
    )(h, pos, wh, bh, wfp, wfx)


def _post_body(m_ref, t_ref, h_ref, bf, wg, bg, wl, bl, o_ref):
    m = m_ref[:, :D].astype(jnp.float32)
    aggr = jnp.where(jnp.isfinite(m), m + t_ref[...] + bf[...], 0.0)
    out = aggr @ wg[...] + bg[...]
    hr = jnp.maximum(h_ref[...] + out, 0.0)
    o_ref[...] = hr @ wl[...] + bl[...]


def _conv_post(m, t, h, p, plin):
    (_wf, bf), = p['f']
    (wg, bg), = p['g']
    (wl, bl), = plin
    return pl.pallas_call(
        _post_body,
        grid=(NB,),
        in_specs=[
            pl.BlockSpec((BLK, 2 * DAI), _rows),
            pl.BlockSpec((BLK, D), _rows),
            pl.BlockSpec((BLK, D), _rows),
            pl.BlockSpec((D,), lambda i: (0,)),
            pl.BlockSpec((D, D), _full), pl.BlockSpec((D,), lambda i: (0,)),
            pl.BlockSpec((D, D), _full), pl.BlockSpec((D,), lambda i: (0,)),
        ],
        out_specs=pl.BlockSpec((BLK, D), _rows),
        out_shape=jax.ShapeDtypeStruct((N, D), jnp.float32),
    )(m, t, h, bf, wg, bg, wl, bl)


def _head_body(h_ref, onehot_ref, wd, bd, o_ref, acc_s, acc_c):
    i = pl.program_id(0)

    @pl.when(i == 0)
    def _init():
        acc_s[...] = jnp.zeros_like(acc_s)
        acc_c[...] = jnp.zeros_like(acc_c)

    oh = onehot_ref[...]
    acc_s[...] += oh.T @ h_ref[...]
    acc_c[...] += jnp.sum(oh, axis=0)[:, None]

    @pl.when(i == NB - 1)
    def _fin():
        pooled = acc_s[...] / jnp.maximum(acc_c[...], 1.0)
        logits = pooled @ wd[...] + bd[...]
        mx = jnp.max(logits, axis=1, keepdims=True)
        sh = logits - mx
        lse = jnp.log(jnp.sum(jnp.exp(sh), axis=1, keepdims=True))
        o_ref[...] = sh - lse


def _head(h, batch, p):
    (wd, bd), = p
    onehot = (batch[:, None] == jnp.arange(G)[None, :]).astype(jnp.float32)
    return pl.pallas_call(
        _head_body,
        grid=(NB,),
        in_specs=[
            pl.BlockSpec((BLK, D), _rows),
            pl.BlockSpec((BLK, G), _rows),
            pl.BlockSpec((D, 40), _full), pl.BlockSpec((40,), lambda i: (0,)),
        ],
        out_specs=pl.BlockSpec((G, 40), _full),
        out_shape=jax.ShapeDtypeStruct((G, 40), jnp.float32),
        scratch_shapes=[pltpu.VMEM((G, D), jnp.float32),
                        pltpu.VMEM((G, 1), jnp.float32)],
    )(h, onehot, wd, bd)


def _sc_segment_max(s_i32, src, dst):
    mesh = plsc.VectorSubcoreMesh(core_axis_name="c", subcore_axis_name="s")
    nvec = CH // 16
    nfv = DAI // 16

    @functools.partial(
        pl.kernel,
        out_type=jax.ShapeDtypeStruct((NPAD, DAI), jnp.int32),
        mesh=mesh,
        compiler_params=pltpu.CompilerParams(needs_layout_passes=False),
        scratch_types=[
            pltpu.VMEM((WPD + 1, DAI), jnp.int32),
            pltpu.VMEM((2 * CH,), jnp.int32),
            pltpu.VMEM((2 * CH,), jnp.int32),
            pltpu.VMEM((CAP,), jnp.int32),
            pltpu.VMEM((CAP,), jnp.int32),
            pltpu.VMEM((2 * SUB, DPI), jnp.int32),
            pltpu.SemaphoreType.DMA,
            pltpu.SemaphoreType.DMA,
            pltpu.SemaphoreType.DMA,
        ],
    )
    def k(s_hbm, src_hbm, dst_hbm, out_hbm,
          acc, sbuf, dbuf, srcc, dlc, rows, sem_s, sem_d, sem_g):
        wid = lax.axis_index("c") * 16 + lax.axis_index("s")
        lo = wid * WPD

        neg = plsc.bitcast(jnp.full((32,), -jnp.inf, dtype=jnp.bfloat16),
                           jnp.int32)

        def init_body(r, carry):
            for f in range(nfv):
                acc[r, pl.ds(f * 16, 16)] = neg
            return carry
        lax.fori_loop(0, WPD + 1, init_body, 0)

        def start_chunk(ch, slot):
            pltpu.async_copy(src_hbm.at[pl.ds(ch * CH, CH)],
                             sbuf.at[pl.ds(slot * CH, CH)], sem_s)
            pltpu.async_copy(dst_hbm.at[pl.ds(ch * CH, CH)],
                             dbuf.at[pl.ds(slot * CH, CH)], sem_d)

        def wait_chunk(ch, slot):
            pltpu.make_async_copy(src_hbm.at[pl.ds(ch * CH, CH)],
                                  sbuf.at[pl.ds(slot * CH, CH)], sem_s).wait()
            pltpu.make_async_copy(dst_hbm.at[pl.ds(ch * CH, CH)],
                                  dbuf.at[pl.ds(slot * CH, CH)], sem_d).wait()

        def start_gather(j, gs):
            pltpu.async_copy(s_hbm.at[srcc.at[pl.ds(j * SUB, SUB)]],
                             rows.at[pl.ds(gs * SUB, SUB)], sem_g)

        def wait_gather(j, gs):
            pltpu.make_async_copy(s_hbm.at[srcc.at[pl.ds(j * SUB, SUB)]],
                                  rows.at[pl.ds(gs * SUB, SUB)], sem_g).wait()

        start_chunk(0, 0)

        lanes = lax.iota(jnp.int32, 16)
        ones = jnp.full((16,), 1, jnp.int32)
        zeros = jnp.full((16,), 0, jnp.int32)

        def chunk_body(ch, carry):
            slot = lax.rem(ch, 2)
            wait_chunk(ch, slot)

            @pl.when(ch + 1 < NCH)
            def _prefetch():
                start_chunk(ch + 1, 1 - slot)

            def filt(v, off):
                d16 = dbuf[pl.ds(slot * CH + v * 16, 16)]
                s16 = sbuf[pl.ds(slot * CH + v * 16, 16)]
                msk = (d16 >= lo) & (d16 < lo + WPD)
                mi = jnp.where(msk, ones, zeros)
                pf = plsc.cumsum(mi)
                pos = off + pf - 1
                plsc.store_scatter(srcc, [pos], s16, mask=msk)
                plsc.store_scatter(dlc, [pos], d16 - lo, mask=msk)
                return off + pf[15]
            off = lax.fori_loop(0, nvec, filt, jnp.int32(0))

            for kk in range(SUB // 16):
                idxp = off + lanes + (kk * 16)
                plsc.store_scatter(srcc, [idxp], zeros)
                plsc.store_scatter(dlc, [idxp],
                                   jnp.full((16,), DUST, jnp.int32))

            nb = lax.shift_right_logical(off + (SUB - 1), 6)

            start_gather(0, 0)

            def gather_body(j, carry2):
                gs = lax.rem(j, 2)
                wait_gather(j, gs)

                @pl.when(j + 1 < nb)
                def _pref2():
                    start_gather(j + 1, 1 - gs)

                def group_body(g, carry3):
                    dl16 = dlc[pl.ds(j * SUB + g * 16, 16)]
                    for l in range(16):
                        dd = dl16[l]
                        row = gs * SUB + g * 16 + l
                        for f in range(nfv):
                            sl = pl.ds(f * 16, 16)
                            a = plsc.bitcast(acc[dd, sl], jnp.bfloat16)
                            r = plsc.bitcast(rows[row, sl], jnp.bfloat16)
                            acc[dd, sl] = plsc.bitcast(jnp.maximum(a, r),
                                                       jnp.int32)
                    return carry3
                lax.fori_loop(0, SUB // 16, group_body, 0)
                return carry2
            lax.fori_loop(0, nb, gather_body, 0)
            return carry
        lax.fori_loop(0, NCH, chunk_body, 0)

        pltpu.sync_copy(acc.at[pl.ds(0, WPD)], out_hbm.at[pl.ds(lo, WPD)])

    return k(s_i32, src, dst)


def kernel(x, edge_index, batch, params):
    src = edge_index[0]
    dst = edge_index[1]
    pos = x
    h = _project(x, params['project'])
    for c, l in (('conv1', 'lin1'), ('conv2', 'lin2'), ('conv3', 'lin3')):
        p = params[c]
        s_bf, t = _conv_pre(h, pos, p)
        s_i32 = lax.bitcast_convert_type(
            s_bf.reshape(N, DPI, 2), jnp.int32)
        m_i32 = _sc_segment_max(s_i32, src, dst)
        m_bf = lax.bitcast_convert_type(
            m_i32, jnp.bfloat16).reshape(NPAD, 2 * DAI)
        h = _conv_post(m_bf, t, h, p, params[l])
    return _head(h, batch, params['decision'])


# speedup vs baseline: 1.7684x; 1.7684x over previous
import functools

import jax
import jax.numpy as jnp
from jax import lax
from jax.experimental import pallas as pl
from jax.experimental.pallas import tpu as pltpu
from jax.experimental.pallas import tpu_sc as plsc

N = 10000
E = 160000
G = 8
D = 300
NB = 25
BLK = N // NB

DPB = 512
DPI = 256
DAI = 160
NW = 32
WPD = 320
NPAD = NW * WPD
DUST = WPD
CH = 2000
NCH = E // CH
SUB = 32
NBUF = 4
CAP = CH + SUB


def _rows(i):
    return (i, 0)


def _full(i):
    return (0, 0)


def _project_body(x_ref, w0, b0, w1, b1, w2, b2, o_ref):
    h = jnp.maximum(x_ref[...] @ w0[...] + b0[...], 0.0)
    h = jnp.maximum(h @ w1[...] + b1[...], 0.0)
    o_ref[...] = h @ w2[...] + b2[...]


def _project(x, p):
    (w0, b0), (w1, b1), (w2, b2) = p
    return pl.pallas_call(
        _project_body,
        grid=(NB,),
        in_specs=[
            pl.BlockSpec((BLK, 3), _rows),
            pl.BlockSpec((3, 64), _full), pl.BlockSpec((64,), lambda i: (0,)),
            pl.BlockSpec((64, 128), _full), pl.BlockSpec((128,), lambda i: (0,)),
            pl.BlockSpec((128, D), _full), pl.BlockSpec((D,), lambda i: (0,)),
        ],
        out_specs=pl.BlockSpec((BLK, D), _rows),
        out_shape=jax.ShapeDtypeStruct((N, D), jnp.float32),
    )(x, w0, b0, w1, b1, w2, b2)


def _pre_body(h_ref, pos_ref, wh, bh, wfp, wfx, s_ref, t_ref):
    h = h_ref[...]
    pos = pos_ref[...]
    delta = h @ wh[...] + bh[...]
    s = pos @ wfp[...] + h @ wfx[...]
    s_ref[...] = jnp.concatenate(
        [s, jnp.zeros((s.shape[0], DPB - D), jnp.float32)],
        axis=1).astype(jnp.bfloat16)
    t_ref[...] = (delta - pos) @ wfp[...]


def _conv_pre(h, pos, p):
    (wh, bh), = p['h']
    (wf, _bf), = p['f']
    wfp = wf[:3]
    wfx = wf[3:]
    return pl.pallas_call(
        _pre_body,
        grid=(NB,),
        in_specs=[
            pl.BlockSpec((BLK, D), _rows),
            pl.BlockSpec((BLK, 3), _rows),
            pl.BlockSpec((D, 3), _full), pl.BlockSpec((3,), lambda i: (0,)),
            pl.BlockSpec((3, D), _full),
            pl.BlockSpec((D, D), _full),
        ],
        out_specs=[pl.BlockSpec((BLK, DPB), _rows), pl.BlockSpec((BLK, D), _rows)],
        out_shape=[jax.ShapeDtypeStruct((N, DPB), jnp.bfloat16),
                   jax.ShapeDtypeStruct((N, D), jnp.float32)],
    )(h, pos, wh, bh, wfp, wfx)


def _post_body(m_ref, t_ref, h_ref, bf, wg, bg, wl, bl, o_ref):
    m = m_ref[:, :D].astype(jnp.float32)
    aggr = jnp.where(jnp.isfinite(m), m + t_ref[...] + bf[...], 0.0)
    out = aggr @ wg[...] + bg[...]
    hr = jnp.maximum(h_ref[...] + out, 0.0)
    o_ref[...] = hr @ wl[...] + bl[...]


def _conv_post(m, t, h, p, plin):
    (_wf, bf), = p['f']
    (wg, bg), = p['g']
    (wl, bl), = plin
    return pl.pallas_call(
        _post_body,
        grid=(NB,),
        in_specs=[
            pl.BlockSpec((BLK, 2 * DAI), _rows),
            pl.BlockSpec((BLK, D), _rows),
            pl.BlockSpec((BLK, D), _rows),
            pl.BlockSpec((D,), lambda i: (0,)),
            pl.BlockSpec((D, D), _full), pl.BlockSpec((D,), lambda i: (0,)),
            pl.BlockSpec((D, D), _full), pl.BlockSpec((D,), lambda i: (0,)),
        ],
        out_specs=pl.BlockSpec((BLK, D), _rows),
        out_shape=jax.ShapeDtypeStruct((N, D), jnp.float32),
    )(m, t, h, bf, wg, bg, wl, bl)


def _head_body(h_ref, onehot_ref, wd, bd, o_ref, acc_s, acc_c):
    i = pl.program_id(0)

    @pl.when(i == 0)
    def _init():
        acc_s[...] = jnp.zeros_like(acc_s)
        acc_c[...] = jnp.zeros_like(acc_c)

    oh = onehot_ref[...]
    acc_s[...] += oh.T @ h_ref[...]
    acc_c[...] += jnp.sum(oh, axis=0)[:, None]

    @pl.when(i == NB - 1)
    def _fin():
        pooled = acc_s[...] / jnp.maximum(acc_c[...], 1.0)
        logits = pooled @ wd[...] + bd[...]
        mx = jnp.max(logits, axis=1, keepdims=True)
        sh = logits - mx
        lse = jnp.log(jnp.sum(jnp.exp(sh), axis=1, keepdims=True))
        o_ref[...] = sh - lse


def _head(h, batch, p):
    (wd, bd), = p
    onehot = (batch[:, None] == jnp.arange(G)[None, :]).astype(jnp.float32)
    return pl.pallas_call(
        _head_body,
        grid=(NB,),
        in_specs=[
            pl.BlockSpec((BLK, D), _rows),
            pl.BlockSpec((BLK, G), _rows),
            pl.BlockSpec((D, 40), _full), pl.BlockSpec((40,), lambda i: (0,)),
        ],
        out_specs=pl.BlockSpec((G, 40), _full),
        out_shape=jax.ShapeDtypeStruct((G, 40), jnp.float32),
        scratch_shapes=[pltpu.VMEM((G, D), jnp.float32),
                        pltpu.VMEM((G, 1), jnp.float32)],
    )(h, onehot, wd, bd)


def _sc_segment_max(s_i32, src, dst):
    mesh = plsc.VectorSubcoreMesh(core_axis_name="c", subcore_axis_name="s")
    nvec = CH // 16
    nfv = DAI // 16

    @functools.partial(
        pl.kernel,
        out_type=jax.ShapeDtypeStruct((NPAD, DAI), jnp.int32),
        mesh=mesh,
        compiler_params=pltpu.CompilerParams(needs_layout_passes=False),
        scratch_types=[
            pltpu.VMEM((WPD + 1, DAI), jnp.int32),
            pltpu.VMEM((2 * CH,), jnp.int32),
            pltpu.VMEM((2 * CH,), jnp.int32),
            pltpu.VMEM((CAP,), jnp.int32),
            pltpu.VMEM((CAP,), jnp.int32),
            pltpu.VMEM((NBUF * SUB, DPI), jnp.int32),
            pltpu.SemaphoreType.DMA,
            pltpu.SemaphoreType.DMA,
            pltpu.SemaphoreType.DMA,
        ],
    )
    def k(s_hbm, src_hbm, dst_hbm, out_hbm,
          acc, sbuf, dbuf, srcc, dlc, rows, sem_s, sem_d, sem_g):
        wid = lax.axis_index("c") * 16 + lax.axis_index("s")
        lo = wid * WPD

        neg = plsc.bitcast(jnp.full((32,), -jnp.inf, dtype=jnp.bfloat16),
                           jnp.int32)

        def init_body(r, carry):
            for f in range(nfv):
                acc[r, pl.ds(f * 16, 16)] = neg
            return carry
        lax.fori_loop(0, WPD + 1, init_body, 0)

        def start_chunk(ch, slot):
            pltpu.async_copy(src_hbm.at[pl.ds(ch * CH, CH)],
                             sbuf.at[pl.ds(slot * CH, CH)], sem_s)
            pltpu.async_copy(dst_hbm.at[pl.ds(ch * CH, CH)],
                             dbuf.at[pl.ds(slot * CH, CH)], sem_d)

        def wait_chunk(ch, slot):
            pltpu.make_async_copy(src_hbm.at[pl.ds(ch * CH, CH)],
                                  sbuf.at[pl.ds(slot * CH, CH)], sem_s).wait()
            pltpu.make_async_copy(dst_hbm.at[pl.ds(ch * CH, CH)],
                                  dbuf.at[pl.ds(slot * CH, CH)], sem_d).wait()

        def start_gather(j, gs):
            pltpu.async_copy(s_hbm.at[srcc.at[pl.ds(j * SUB, SUB)]],
                             rows.at[pl.ds(gs * SUB, SUB)], sem_g)

        def wait_gather(j, gs):
            pltpu.make_async_copy(s_hbm.at[srcc.at[pl.ds(j * SUB, SUB)]],
                                  rows.at[pl.ds(gs * SUB, SUB)], sem_g).wait()

        start_chunk(0, 0)

        lanes = lax.iota(jnp.int32, 16)
        ones = jnp.full((16,), 1, jnp.int32)
        zeros = jnp.full((16,), 0, jnp.int32)

        def chunk_body(ch, carry):
            slot = lax.rem(ch, 2)
            wait_chunk(ch, slot)

            @pl.when(ch + 1 < NCH)
            def _prefetch():
                start_chunk(ch + 1, 1 - slot)

            def filt(v, off):
                d16 = dbuf[pl.ds(slot * CH + v * 16, 16)]
                s16 = sbuf[pl.ds(slot * CH + v * 16, 16)]
                msk = (d16 >= lo) & (d16 < lo + WPD)
                mi = jnp.where(msk, ones, zeros)
                pf = plsc.cumsum(mi)
                pos = off + pf - 1
                plsc.store_scatter(srcc, [pos], s16, mask=msk)
                plsc.store_scatter(dlc, [pos], d16 - lo, mask=msk)
                return off + pf[15]
            off = lax.fori_loop(0, nvec, filt, jnp.int32(0))

            for kk in range(SUB // 16):
                idxp = off + lanes + (kk * 16)
                plsc.store_scatter(srcc, [idxp], zeros)
                plsc.store_scatter(dlc, [idxp],
                                   jnp.full((16,), DUST, jnp.int32))

            nb = lax.shift_right_logical(off + (SUB - 1), 5)

            for jj in range(NBUF - 1):
                @pl.when(jj < nb)
                def _prime():
                    start_gather(jj, jj)

            def gather_body(j, carry2):
                gs = lax.rem(j, NBUF)
                wait_gather(j, gs)

                @pl.when(j + NBUF - 1 < nb)
                def _pref2():
                    start_gather(j + NBUF - 1, lax.rem(j + NBUF - 1, NBUF))

                def group_body(g, carry3):
                    dl16 = dlc[pl.ds(j * SUB + g * 16, 16)]
                    for l in range(16):
                        dd = dl16[l]
                        row = gs * SUB + g * 16 + l
                        for f in range(nfv):
                            sl = pl.ds(f * 16, 16)
                            a = plsc.bitcast(acc[dd, sl], jnp.bfloat16)
                            r = plsc.bitcast(rows[row, sl], jnp.bfloat16)
                            acc[dd, sl] = plsc.bitcast(jnp.maximum(a, r),
                                                       jnp.int32)
                    return carry3
                lax.fori_loop(0, SUB // 16, group_body, 0)
                return carry2
            lax.fori_loop(0, nb, gather_body, 0)
            return carry
        lax.fori_loop(0, NCH, chunk_body, 0)

        pltpu.sync_copy(acc.at[pl.ds(0, WPD)], out_hbm.at[pl.ds(lo, WPD)])

    return k(s_i32, src, dst)


def kernel(x, edge_index, batch, params):
    src = edge_index[0]
    dst = edge_index[1]
    pos = x
    h = _project(x, params['project'])
    for c, l in (('conv1', 'lin1'), ('conv2', 'lin2'), ('conv3', 'lin3')):
        p = params[c]
        s_bf, t = _conv_pre(h, pos, p)
        s_i32 = lax.bitcast_convert_type(
            s_bf.reshape(N, DPI, 2), jnp.int32)
        m_i32 = _sc_segment_max(s_i32, src, dst)
        m_bf = lax.bitcast_convert_type(
            m_i32, jnp.bfloat16).reshape(NPAD, 2 * DAI)
        h = _conv_post(m_bf, t, h, p, params[l])
    return _head(h, batch, params['decision'])


# speedup vs baseline: 2.8349x; 1.6031x over previous
import functools

import jax
import jax.numpy as jnp
from jax import lax
from jax.experimental import pallas as pl
from jax.experimental.pallas import tpu as pltpu
from jax.experimental.pallas import tpu_sc as plsc

N = 10000
E = 160000
G = 8
D = 300
NB = 25
BLK = N // NB

DPB = 512
DPI = 256
DAI = 160
NW = 32
WPD = 320
NPAD = NW * WPD
DUST = WPD
CH = 2000
NCH = E // CH
SUB = 16
NBUF = 8
CAP = CH + SUB


def _rows(i):
    return (i, 0)


def _full(i):
    return (0, 0)


def _project_body(x_ref, w0, b0, w1, b1, w2, b2, o_ref):
    h = jnp.maximum(x_ref[...] @ w0[...] + b0[...], 0.0)
    h = jnp.maximum(h @ w1[...] + b1[...], 0.0)
    o_ref[...] = h @ w2[...] + b2[...]


def _project(x, p):
    (w0, b0), (w1, b1), (w2, b2) = p
    return pl.pallas_call(
        _project_body,
        grid=(NB,),
        in_specs=[
            pl.BlockSpec((BLK, 3), _rows),
            pl.BlockSpec((3, 64), _full), pl.BlockSpec((64,), lambda i: (0,)),
            pl.BlockSpec((64, 128), _full), pl.BlockSpec((128,), lambda i: (0,)),
            pl.BlockSpec((128, D), _full), pl.BlockSpec((D,), lambda i: (0,)),
        ],
        out_specs=pl.BlockSpec((BLK, D), _rows),
        out_shape=jax.ShapeDtypeStruct((N, D), jnp.float32),
    )(x, w0, b0, w1, b1, w2, b2)


def _pre_body(h_ref, pos_ref, wh, bh, wfp, wfx, s_ref, t_ref):
    h = h_ref[...]
    pos = pos_ref[...]
    delta = h @ wh[...] + bh[...]
    s = pos @ wfp[...] + h @ wfx[...]
    s_ref[...] = jnp.concatenate(
        [s, jnp.zeros((s.shape[0], DPB - D), jnp.float32)],
        axis=1).astype(jnp.bfloat16)
    t_ref[...] = (delta - pos) @ wfp[...]


def _conv_pre(h, pos, p):
    (wh, bh), = p['h']
    (wf, _bf), = p['f']
    wfp = wf[:3]
    wfx = wf[3:]
    return pl.pallas_call(
        _pre_body,
        grid=(NB,),
        in_specs=[
            pl.BlockSpec((BLK, D), _rows),
            pl.BlockSpec((BLK, 3), _rows),
            pl.BlockSpec((D, 3), _full), pl.BlockSpec((3,), lambda i: (0,)),
            pl.BlockSpec((3, D), _full),
            pl.BlockSpec((D, D), _full),
        ],
        out_specs=[pl.BlockSpec((BLK, DPB), _rows), pl.BlockSpec((BLK, D), _rows)],
        out_shape=[jax.ShapeDtypeStruct((N, DPB), jnp.bfloat16),
                   jax.ShapeDtypeStruct((N, D), jnp.float32)],
    )(h, pos, wh, bh, wfp, wfx)


def _post_body(m_ref, t_ref, h_ref, bf, wg, bg, wl, bl, o_ref):
    m = m_ref[:, :D].astype(jnp.float32)
    aggr = jnp.where(jnp.isfinite(m), m + t_ref[...] + bf[...], 0.0)
    out = aggr @ wg[...] + bg[...]
    hr = jnp.maximum(h_ref[...] + out, 0.0)
    o_ref[...] = hr @ wl[...] + bl[...]


def _conv_post(m, t, h, p, plin):
    (_wf, bf), = p['f']
    (wg, bg), = p['g']
    (wl, bl), = plin
    return pl.pallas_call(
        _post_body,
        grid=(NB,),
        in_specs=[
            pl.BlockSpec((BLK, 2 * DAI), _rows),
            pl.BlockSpec((BLK, D), _rows),
            pl.BlockSpec((BLK, D), _rows),
            pl.BlockSpec((D,), lambda i: (0,)),
            pl.BlockSpec((D, D), _full), pl.BlockSpec((D,), lambda i: (0,)),
            pl.BlockSpec((D, D), _full), pl.BlockSpec((D,), lambda i: (0,)),
        ],
        out_specs=pl.BlockSpec((BLK, D), _rows),
        out_shape=jax.ShapeDtypeStruct((N, D), jnp.float32),
    )(m, t, h, bf, wg, bg, wl, bl)


def _head_body(h_ref, onehot_ref, wd, bd, o_ref, acc_s, acc_c):
    i = pl.program_id(0)

    @pl.when(i == 0)
    def _init():
        acc_s[...] = jnp.zeros_like(acc_s)
        acc_c[...] = jnp.zeros_like(acc_c)

    oh = onehot_ref[...]
    acc_s[...] += oh.T @ h_ref[...]
    acc_c[...] += jnp.sum(oh, axis=0)[:, None]

    @pl.when(i == NB - 1)
    def _fin():
        pooled = acc_s[...] / jnp.maximum(acc_c[...], 1.0)
        logits = pooled @ wd[...] + bd[...]
        mx = jnp.max(logits, axis=1, keepdims=True)
        sh = logits - mx
        lse = jnp.log(jnp.sum(jnp.exp(sh), axis=1, keepdims=True))
        o_ref[...] = sh - lse


def _head(h, batch, p):
    (wd, bd), = p
    onehot = (batch[:, None] == jnp.arange(G)[None, :]).astype(jnp.float32)
    return pl.pallas_call(
        _head_body,
        grid=(NB,),
        in_specs=[
            pl.BlockSpec((BLK, D), _rows),
            pl.BlockSpec((BLK, G), _rows),
            pl.BlockSpec((D, 40), _full), pl.BlockSpec((40,), lambda i: (0,)),
        ],
        out_specs=pl.BlockSpec((G, 40), _full),
        out_shape=jax.ShapeDtypeStruct((G, 40), jnp.float32),
        scratch_shapes=[pltpu.VMEM((G, D), jnp.float32),
                        pltpu.VMEM((G, 1), jnp.float32)],
    )(h, onehot, wd, bd)


def _sc_segment_max(s_i32, src, dst):
    mesh = plsc.VectorSubcoreMesh(core_axis_name="c", subcore_axis_name="s")
    nvec = CH // 16
    nfv = DAI // 16

    @functools.partial(
        pl.kernel,
        out_type=jax.ShapeDtypeStruct((NPAD, DAI), jnp.int32),
        mesh=mesh,
        compiler_params=pltpu.CompilerParams(needs_layout_passes=False),
        scratch_types=[
            pltpu.VMEM((WPD + 1, DAI), jnp.int32),
            pltpu.VMEM((2 * CH,), jnp.int32),
            pltpu.VMEM((2 * CH,), jnp.int32),
            pltpu.VMEM((CAP,), jnp.int32),
            pltpu.VMEM((CAP,), jnp.int32),
            pltpu.VMEM((NBUF * SUB, DPI), jnp.int32),
            pltpu.SemaphoreType.DMA,
            pltpu.SemaphoreType.DMA,
            pltpu.SemaphoreType.DMA,
        ],
    )
    def k(s_hbm, src_hbm, dst_hbm, out_hbm,
          acc, sbuf, dbuf, srcc, dlc, rows, sem_s, sem_d, sem_g):
        wid = lax.axis_index("c") * 16 + lax.axis_index("s")
        lo = wid * WPD

        neg = plsc.bitcast(jnp.full((32,), -jnp.inf, dtype=jnp.bfloat16),
                           jnp.int32)

        def init_body(r, carry):
            for f in range(nfv):
                acc[r, pl.ds(f * 16, 16)] = neg
            return carry
        lax.fori_loop(0, WPD + 1, init_body, 0)

        def start_chunk(ch, slot):
            pltpu.async_copy(src_hbm.at[pl.ds(ch * CH, CH)],
                             sbuf.at[pl.ds(slot * CH, CH)], sem_s)
            pltpu.async_copy(dst_hbm.at[pl.ds(ch * CH, CH)],
                             dbuf.at[pl.ds(slot * CH, CH)], sem_d)

        def wait_chunk(ch, slot):
            pltpu.make_async_copy(src_hbm.at[pl.ds(ch * CH, CH)],
                                  sbuf.at[pl.ds(slot * CH, CH)], sem_s).wait()
            pltpu.make_async_copy(dst_hbm.at[pl.ds(ch * CH, CH)],
                                  dbuf.at[pl.ds(slot * CH, CH)], sem_d).wait()

        def start_gather(j, gs):
            pltpu.async_copy(s_hbm.at[srcc.at[pl.ds(j * SUB, SUB)]],
                             rows.at[pl.ds(gs * SUB, SUB)], sem_g)

        def wait_gather(j, gs):
            pltpu.make_async_copy(s_hbm.at[srcc.at[pl.ds(j * SUB, SUB)]],
                                  rows.at[pl.ds(gs * SUB, SUB)], sem_g).wait()

        start_chunk(0, 0)

        lanes = lax.iota(jnp.int32, 16)
        ones = jnp.full((16,), 1, jnp.int32)
        zeros = jnp.full((16,), 0, jnp.int32)

        def chunk_body(ch, carry):
            slot = lax.rem(ch, 2)
            wait_chunk(ch, slot)

            @pl.when(ch + 1 < NCH)
            def _prefetch():
                start_chunk(ch + 1, 1 - slot)

            def filt(v, off):
                d16 = dbuf[pl.ds(slot * CH + v * 16, 16)]
                s16 = sbuf[pl.ds(slot * CH + v * 16, 16)]
                msk = (d16 >= lo) & (d16 < lo + WPD)
                mi = jnp.where(msk, ones, zeros)
                pf = plsc.cumsum(mi)
                pos = off + pf - 1
                plsc.store_scatter(srcc, [pos], s16, mask=msk)
                plsc.store_scatter(dlc, [pos], d16 - lo, mask=msk)
                return off + pf[15]
            off = lax.fori_loop(0, nvec, filt, jnp.int32(0))

            for kk in range(SUB // 16):
                idxp = off + lanes + (kk * 16)
                plsc.store_scatter(srcc, [idxp], zeros)
                plsc.store_scatter(dlc, [idxp],
                                   jnp.full((16,), DUST, jnp.int32))

            nb = lax.shift_right_logical(off + (SUB - 1), 4)

            for jj in range(NBUF - 1):
                @pl.when(jj < nb)
                def _prime():
                    start_gather(jj, jj)

            def gather_body(j, carry2):
                gs = lax.rem(j, NBUF)
                wait_gather(j, gs)

                @pl.when(j + NBUF - 1 < nb)
                def _pref2():
                    start_gather(j + NBUF - 1, lax.rem(j + NBUF - 1, NBUF))

                def group_body(g, carry3):
                    dl16 = dlc[pl.ds(j * SUB + g * 16, 16)]
                    for l in range(16):
                        dd = dl16[l]
                        row = gs * SUB + g * 16 + l
                        for f in range(nfv):
                            sl = pl.ds(f * 16, 16)
                            a = plsc.bitcast(acc[dd, sl], jnp.bfloat16)
                            r = plsc.bitcast(rows[row, sl], jnp.bfloat16)
                            acc[dd, sl] = plsc.bitcast(jnp.maximum(a, r),
                                                       jnp.int32)
                    return carry3
                lax.fori_loop(0, SUB // 16, group_body, 0)
                return carry2
            lax.fori_loop(0, nb, gather_body, 0)
            return carry
        lax.fori_loop(0, NCH, chunk_body, 0)

        pltpu.sync_copy(acc.at[pl.ds(0, WPD)], out_hbm.at[pl.ds(lo, WPD)])

    return k(s_i32, src, dst)


def kernel(x, edge_index, batch, params):
    src = edge_index[0]
    dst = edge_index[1]
    pos = x
    h = _project(x, params['project'])
    for c, l in (('conv1', 'lin1'), ('conv2', 'lin2'), ('conv3', 'lin3')):
        p = params[c]
        s_bf, t = _conv_pre(h, pos, p)
        s_i32 = lax.bitcast_convert_type(
            s_bf.reshape(N, DPI, 2), jnp.int32)
        m_i32 = _sc_segment_max(s_i32, src, dst)
        m_bf = lax.bitcast_convert_type(
            m_i32, jnp.bfloat16).reshape(NPAD, 2 * DAI)
        h = _conv_post(m_bf, t, h, p, params[l])
    return _head(h, batch, params['decision'])


# speedup vs baseline: 5.4191x; 1.9116x over previous
import functools

import jax
import jax.numpy as jnp
from jax import lax
from jax.experimental import pallas as pl
from jax.experimental.pallas import tpu as pltpu
from jax.experimental.pallas import tpu_sc as plsc

N = 10000
E = 160000
G = 8
D = 300
NB = 25
BLK = N // NB

DPB = 512
DPI = 256
DAI = 160
NW = 32
WPD = 320
NPAD = NW * WPD
DUST = WPD
CH = 2000
NCH = E // CH
SUB = 8
NBUF = 16
CAP = CH + SUB


def _rows(i):
    return (i, 0)


def _full(i):
    return (0, 0)


def _project_body(x_ref, w0, b0, w1, b1, w2, b2, o_ref):
    h = jnp.maximum(x_ref[...] @ w0[...] + b0[...], 0.0)
    h = jnp.maximum(h @ w1[...] + b1[...], 0.0)
    o_ref[...] = h @ w2[...] + b2[...]


def _project(x, p):
    (w0, b0), (w1, b1), (w2, b2) = p
    return pl.pallas_call(
        _project_body,
        grid=(NB,),
        in_specs=[
            pl.BlockSpec((BLK, 3), _rows),
            pl.BlockSpec((3, 64), _full), pl.BlockSpec((64,), lambda i: (0,)),
            pl.BlockSpec((64, 128), _full), pl.BlockSpec((128,), lambda i: (0,)),
            pl.BlockSpec((128, D), _full), pl.BlockSpec((D,), lambda i: (0,)),
        ],
        out_specs=pl.BlockSpec((BLK, D), _rows),
        out_shape=jax.ShapeDtypeStruct((N, D), jnp.float32),
    )(x, w0, b0, w1, b1, w2, b2)


def _pre_body(h_ref, pos_ref, wh, bh, wfp, wfx, s_ref, t_ref):
    h = h_ref[...]
    pos = pos_ref[...]
    delta = h @ wh[...] + bh[...]
    s = pos @ wfp[...] + h @ wfx[...]
    s_ref[...] = jnp.concatenate(
        [s, jnp.zeros((s.shape[0], DPB - D), jnp.float32)],
        axis=1).astype(jnp.bfloat16)
    t_ref[...] = (delta - pos) @ wfp[...]


def _conv_pre(h, pos, p):
    (wh, bh), = p['h']
    (wf, _bf), = p['f']
    wfp = wf[:3]
    wfx = wf[3:]
    return pl.pallas_call(
        _pre_body,
        grid=(NB,),
        in_specs=[
            pl.BlockSpec((BLK, D), _rows),
            pl.BlockSpec((BLK, 3), _rows),
            pl.BlockSpec((D, 3), _full), pl.BlockSpec((3,), lambda i: (0,)),
            pl.BlockSpec((3, D), _full),
            pl.BlockSpec((D, D), _full),
        ],
        out_specs=[pl.BlockSpec((BLK, DPB), _rows), pl.BlockSpec((BLK, D), _rows)],
        out_shape=[jax.ShapeDtypeStruct((N, DPB), jnp.bfloat16),
                   jax.ShapeDtypeStruct((N, D), jnp.float32)],
    )(h, pos, wh, bh, wfp, wfx)


def _post_body(m_ref, t_ref, h_ref, bf, wg, bg, wl, bl, o_ref):
    m = m_ref[:, :D].astype(jnp.float32)
    aggr = jnp.where(jnp.isfinite(m), m + t_ref[...] + bf[...], 0.0)
    out = aggr @ wg[...] + bg[...]
    hr = jnp.maximum(h_ref[...] + out, 0.0)
    o_ref[...] = hr @ wl[...] + bl[...]


def _conv_post(m, t, h, p, plin):
    (_wf, bf), = p['f']
    (wg, bg), = p['g']
    (wl, bl), = plin
    return pl.pallas_call(
        _post_body,
        grid=(NB,),
        in_specs=[
            pl.BlockSpec((BLK, 2 * DAI), _rows),
            pl.BlockSpec((BLK, D), _rows),
            pl.BlockSpec((BLK, D), _rows),
            pl.BlockSpec((D,), lambda i: (0,)),
            pl.BlockSpec((D, D), _full), pl.BlockSpec((D,), lambda i: (0,)),
            pl.BlockSpec((D, D), _full), pl.BlockSpec((D,), lambda i: (0,)),
        ],
        out_specs=pl.BlockSpec((BLK, D), _rows),
        out_shape=jax.ShapeDtypeStruct((N, D), jnp.float32),
    )(m, t, h, bf, wg, bg, wl, bl)


def _head_body(h_ref, onehot_ref, wd, bd, o_ref, acc_s, acc_c):
    i = pl.program_id(0)

    @pl.when(i == 0)
    def _init():
        acc_s[...] = jnp.zeros_like(acc_s)
        acc_c[...] = jnp.zeros_like(acc_c)

    oh = onehot_ref[...]
    acc_s[...] += oh.T @ h_ref[...]
    acc_c[...] += jnp.sum(oh, axis=0)[:, None]

    @pl.when(i == NB - 1)
    def _fin():
        pooled = acc_s[...] / jnp.maximum(acc_c[...], 1.0)
        logits = pooled @ wd[...] + bd[...]
        mx = jnp.max(logits, axis=1, keepdims=True)
        sh = logits - mx
        lse = jnp.log(jnp.sum(jnp.exp(sh), axis=1, keepdims=True))
        o_ref[...] = sh - lse


def _head(h, batch, p):
    (wd, bd), = p
    onehot = (batch[:, None] == jnp.arange(G)[None, :]).astype(jnp.float32)
    return pl.pallas_call(
        _head_body,
        grid=(NB,),
        in_specs=[
            pl.BlockSpec((BLK, D), _rows),
            pl.BlockSpec((BLK, G), _rows),
            pl.BlockSpec((D, 40), _full), pl.BlockSpec((40,), lambda i: (0,)),
        ],
        out_specs=pl.BlockSpec((G, 40), _full),
        out_shape=jax.ShapeDtypeStruct((G, 40), jnp.float32),
        scratch_shapes=[pltpu.VMEM((G, D), jnp.float32),
                        pltpu.VMEM((G, 1), jnp.float32)],
    )(h, onehot, wd, bd)


def _sc_segment_max(s_i32, src, dst):
    mesh = plsc.VectorSubcoreMesh(core_axis_name="c", subcore_axis_name="s")
    nvec = CH // 16
    nfv = DAI // 16

    @functools.partial(
        pl.kernel,
        out_type=jax.ShapeDtypeStruct((NPAD, DAI), jnp.int32),
        mesh=mesh,
        compiler_params=pltpu.CompilerParams(needs_layout_passes=False),
        scratch_types=[
            pltpu.VMEM((WPD + 1, DAI), jnp.int32),
            pltpu.VMEM((2 * CH,), jnp.int32),
            pltpu.VMEM((2 * CH,), jnp.int32),
            pltpu.VMEM((CAP,), jnp.int32),
            pltpu.VMEM((CAP,), jnp.int32),
            pltpu.VMEM((NBUF * SUB, DPI), jnp.int32),
            pltpu.SemaphoreType.DMA,
            pltpu.SemaphoreType.DMA,
            pltpu.SemaphoreType.DMA,
        ],
    )
    def k(s_hbm, src_hbm, dst_hbm, out_hbm,
          acc, sbuf, dbuf, srcc, dlc, rows, sem_s, sem_d, sem_g):
        wid = lax.axis_index("c") * 16 + lax.axis_index("s")
        lo = wid * WPD

        neg = plsc.bitcast(jnp.full((32,), -jnp.inf, dtype=jnp.bfloat16),
                           jnp.int32)

        def init_body(r, carry):
            for f in range(nfv):
                acc[r, pl.ds(f * 16, 16)] = neg
            return carry
        lax.fori_loop(0, WPD + 1, init_body, 0)

        def start_chunk(ch, slot):
            pltpu.async_copy(src_hbm.at[pl.ds(ch * CH, CH)],
                             sbuf.at[pl.ds(slot * CH, CH)], sem_s)
            pltpu.async_copy(dst_hbm.at[pl.ds(ch * CH, CH)],
                             dbuf.at[pl.ds(slot * CH, CH)], sem_d)

        def wait_chunk(ch, slot):
            pltpu.make_async_copy(src_hbm.at[pl.ds(ch * CH, CH)],
                                  sbuf.at[pl.ds(slot * CH, CH)], sem_s).wait()
            pltpu.make_async_copy(dst_hbm.at[pl.ds(ch * CH, CH)],
                                  dbuf.at[pl.ds(slot * CH, CH)], sem_d).wait()

        def start_gather(j, gs):
            pltpu.async_copy(s_hbm.at[srcc.at[pl.ds(j * SUB, SUB)]],
                             rows.at[pl.ds(gs * SUB, SUB)], sem_g)

        def wait_gather(j, gs):
            pltpu.make_async_copy(s_hbm.at[srcc.at[pl.ds(j * SUB, SUB)]],
                                  rows.at[pl.ds(gs * SUB, SUB)], sem_g).wait()

        start_chunk(0, 0)

        lanes = lax.iota(jnp.int32, 16)
        ones = jnp.full((16,), 1, jnp.int32)
        zeros = jnp.full((16,), 0, jnp.int32)

        def chunk_body(ch, carry):
            slot = lax.rem(ch, 2)
            wait_chunk(ch, slot)

            @pl.when(ch + 1 < NCH)
            def _prefetch():
                start_chunk(ch + 1, 1 - slot)

            def filt(v, off):
                d16 = dbuf[pl.ds(slot * CH + v * 16, 16)]
                s16 = sbuf[pl.ds(slot * CH + v * 16, 16)]
                msk = (d16 >= lo) & (d16 < lo + WPD)
                mi = jnp.where(msk, ones, zeros)
                pf = plsc.cumsum(mi)
                pos = off + pf - 1
                plsc.store_scatter(srcc, [pos], s16, mask=msk)
                plsc.store_scatter(dlc, [pos], d16 - lo, mask=msk)
                return off + pf[15]
            off = lax.fori_loop(0, nvec, filt, jnp.int32(0))

            for kk in range(SUB // 16):
                idxp = off + lanes + (kk * 16)
                plsc.store_scatter(srcc, [idxp], zeros)
                plsc.store_scatter(dlc, [idxp],
                                   jnp.full((16,), DUST, jnp.int32))

            nb = lax.shift_right_logical(off + (SUB - 1), 3)

            for jj in range(NBUF - 1):
                @pl.when(jj < nb)
                def _prime():
                    start_gather(jj, jj)

            def gather_body(j, carry2):
                gs = lax.rem(j, NBUF)
                wait_gather(j, gs)

                @pl.when(j + NBUF - 1 < nb)
                def _pref2():
                    start_gather(j + NBUF - 1, lax.rem(j + NBUF - 1, NBUF))

                def group_body(g, carry3):
                    dl16 = dlc[pl.ds(j * SUB + g * 16, 16)]
                    for l in range(16):
                        dd = dl16[l]
                        row = gs * SUB + g * 16 + l
                        for f in range(nfv):
                            sl = pl.ds(f * 16, 16)
                            a = plsc.bitcast(acc[dd, sl], jnp.bfloat16)
                            r = plsc.bitcast(rows[row, sl], jnp.bfloat16)
                            acc[dd, sl] = plsc.bitcast(jnp.maximum(a, r),
                                                       jnp.int32)
                    return carry3
                lax.fori_loop(0, SUB // 16, group_body, 0)
                return carry2
            lax.fori_loop(0, nb, gather_body, 0)
            return carry
        lax.fori_loop(0, NCH, chunk_body, 0)

        pltpu.sync_copy(acc.at[pl.ds(0, WPD)], out_hbm.at[pl.ds(lo, WPD)])

    return k(s_i32, src, dst)


def kernel(x, edge_index, batch, params):
    src = edge_index[0]
    dst = edge_index[1]
    pos = x
    h = _project(x, params['project'])
    for c, l in (('conv1', 'lin1'), ('conv2', 'lin2'), ('conv3', 'lin3')):
        p = params[c]
        s_bf, t = _conv_pre(h, pos, p)
        s_i32 = lax.bitcast_convert_type(
            s_bf.reshape(N, DPI, 2), jnp.int32)
        m_i32 = _sc_segment_max(s_i32, src, dst)
        m_bf = lax.bitcast_convert_type(
            m_i32, jnp.bfloat16).reshape(NPAD, 2 * DAI)
        h = _conv_post(m_bf, t, h, p, params[l])
    return _head(h, batch, params['decision'])
